# Initial kernel scaffold; baseline (speedup 1.0000x reference)
import jax
import jax.numpy as jnp
from jax.experimental import pallas as pl


def kernel(x, w0, w1, w2):
    raise NotImplementedError("write your pallas kernel here")


# speedup vs baseline: 86.1122x; 86.1122x over previous
import functools

import jax
import jax.numpy as jnp
from jax.experimental import pallas as pl

NETS = 100
D = 64
BATCH = 1024
P = 2
GRID = NETS // P


def _block_diag(w_stacked):
    zeros = jnp.zeros((D, D), dtype=w_stacked.dtype)
    rows = []
    for p in range(P):
        blk = w_stacked[p * D:(p + 1) * D, :]
        row = [blk if q == p else zeros for q in range(P)]
        rows.append(jnp.concatenate(row, axis=1))
    return jnp.concatenate(rows, axis=0)


def _mm(a, b):
    return jax.lax.dot_general(
        a, b, (((1,), (0,)), ((), ())),
        precision=jax.lax.Precision.HIGHEST,
        preferred_element_type=jnp.float32)


def _step(x_ref, w0_ref, w1_ref, w2_ref, out_ref):
    b0 = _block_diag(w0_ref[...])
    b1 = _block_diag(w1_ref[...])
    b2 = _block_diag(w2_ref[...])
    m = _mm(b2, _mm(b1, b0))
    out_ref[...] = _mm(m, x_ref[...])


@jax.jit
def kernel(x, w0, w1, w2):
    w0m = w0.reshape(NETS * D, D)
    w1m = w1.reshape(NETS * D, D)
    w2m = w2.reshape(NETS * D, D)
    wspec = pl.BlockSpec((P * D, D), lambda i: (i, 0))
    xspec = pl.BlockSpec((P * D, BATCH), lambda i: (i, 0))
    return pl.pallas_call(
        _step,
        grid=(GRID,),
        in_specs=[xspec, wspec, wspec, wspec],
        out_specs=xspec,
        out_shape=jax.ShapeDtypeStruct((NETS * D, BATCH), jnp.float32),
    )(x, w0m, w1m, w2m)


# speedup vs baseline: 99.8692x; 1.1598x over previous
import functools

import jax
import jax.numpy as jnp
from jax.experimental import pallas as pl

NETS = 100
D = 64
BATCH = 1024
P = 2
GRID = NETS // P


def _block_diag(w_stacked):
    zeros = jnp.zeros((D, D), dtype=w_stacked.dtype)
    rows = []
    for p in range(P):
        blk = w_stacked[p * D:(p + 1) * D, :]
        row = [blk if q == p else zeros for q in range(P)]
        rows.append(jnp.concatenate(row, axis=1))
    return jnp.concatenate(rows, axis=0)


def _mm(a, b, prec):
    return jax.lax.dot_general(
        a, b, (((1,), (0,)), ((), ())),
        precision=prec,
        preferred_element_type=jnp.float32)


def _step(x_ref, w0_ref, w1_ref, w2_ref, out_ref):
    hi = jax.lax.Precision.HIGHEST
    b0 = _block_diag(w0_ref[...])
    b1 = _block_diag(w1_ref[...])
    b2 = _block_diag(w2_ref[...])
    m = _mm(b2, _mm(b1, b0, hi), hi)
    out_ref[...] = _mm(m, x_ref[...], jax.lax.Precision.DEFAULT)


@jax.jit
def kernel(x, w0, w1, w2):
    w0m = w0.reshape(NETS * D, D)
    w1m = w1.reshape(NETS * D, D)
    w2m = w2.reshape(NETS * D, D)
    wspec = pl.BlockSpec((P * D, D), lambda i: (i, 0))
    xspec = pl.BlockSpec((P * D, BATCH), lambda i: (i, 0))
    return pl.pallas_call(
        _step,
        grid=(GRID,),
        in_specs=[xspec, wspec, wspec, wspec],
        out_specs=xspec,
        out_shape=jax.ShapeDtypeStruct((NETS * D, BATCH), jnp.float32),
    )(x, w0m, w1m, w2m)


# speedup vs baseline: 108.1169x; 1.0826x over previous
import functools

import jax
import jax.numpy as jnp
from jax.experimental import pallas as pl

NETS = 100
D = 64
BATCH = 1024
P = 2
GRID = NETS // P


def _block_diag(w_stacked):
    zeros = jnp.zeros((D, D), dtype=w_stacked.dtype)
    rows = []
    for p in range(P):
        blk = w_stacked[p * D:(p + 1) * D, :]
        row = [blk if q == p else zeros for q in range(P)]
        rows.append(jnp.concatenate(row, axis=1))
    return jnp.concatenate(rows, axis=0)


def _mm(a, b, prec):
    return jax.lax.dot_general(
        a, b, (((1,), (0,)), ((), ())),
        precision=prec,
        preferred_element_type=jnp.float32)


def _step(x_ref, w0_ref, w1_ref, w2_ref, out_ref):
    df = jax.lax.Precision.DEFAULT
    b0 = _block_diag(w0_ref[...])
    b1 = _block_diag(w1_ref[...])
    b2 = _block_diag(w2_ref[...])
    m = _mm(b2, _mm(b1, b0, df), df)
    out_ref[...] = _mm(m.astype(jnp.bfloat16), x_ref[...].astype(jnp.bfloat16), df)


@jax.jit
def kernel(x, w0, w1, w2):
    w0m = w0.reshape(NETS * D, D)
    w1m = w1.reshape(NETS * D, D)
    w2m = w2.reshape(NETS * D, D)
    wspec = pl.BlockSpec((P * D, D), lambda i: (i, 0))
    xspec = pl.BlockSpec((P * D, BATCH), lambda i: (i, 0))
    return pl.pallas_call(
        _step,
        grid=(GRID,),
        in_specs=[xspec, wspec, wspec, wspec],
        out_specs=xspec,
        out_shape=jax.ShapeDtypeStruct((NETS * D, BATCH), jnp.float32),
    )(x, w0m, w1m, w2m)


# speedup vs baseline: 172.8115x; 1.5984x over previous
import jax
import jax.numpy as jnp
from jax.experimental import pallas as pl

NETS = 100
D = 64
BATCH = 1024
P = 2
Q = 5
GRID = NETS // (P * Q)


def _block_diag(w_stacked):
    zeros = jnp.zeros((D, D), dtype=w_stacked.dtype)
    rows = []
    for p in range(P):
        blk = w_stacked[p * D:(p + 1) * D, :]
        row = [blk if q == p else zeros for q in range(P)]
        rows.append(jnp.concatenate(row, axis=1))
    return jnp.concatenate(rows, axis=0)


def _mm(a, b):
    return jax.lax.dot_general(
        a, b, (((1,), (0,)), ((), ())),
        precision=jax.lax.Precision.DEFAULT,
        preferred_element_type=jnp.float32)


def _step(x_ref, w0_ref, w1_ref, w2_ref, out_ref):
    for q in range(Q):
        sl = pl.ds(q * P * D, P * D)
        b0 = _block_diag(w0_ref[sl, :])
        b1 = _block_diag(w1_ref[sl, :])
        b2 = _block_diag(w2_ref[sl, :])
        m = _mm(b2, _mm(b1, b0))
        out_ref[sl, :] = _mm(m.astype(jnp.bfloat16),
                             x_ref[sl, :].astype(jnp.bfloat16))


@jax.jit
def kernel(x, w0, w1, w2):
    w0m = w0.reshape(NETS * D, D)
    w1m = w1.reshape(NETS * D, D)
    w2m = w2.reshape(NETS * D, D)
    wspec = pl.BlockSpec((Q * P * D, D), lambda i: (i, 0))
    xspec = pl.BlockSpec((Q * P * D, BATCH), lambda i: (i, 0))
    return pl.pallas_call(
        _step,
        grid=(GRID,),
        in_specs=[xspec, wspec, wspec, wspec],
        out_specs=xspec,
        out_shape=jax.ShapeDtypeStruct((NETS * D, BATCH), jnp.float32),
    )(x, w0m, w1m, w2m)


# speedup vs baseline: 179.1849x; 1.0369x over previous
import jax
import jax.numpy as jnp
from jax.experimental import pallas as pl

NETS = 100
D = 64
BATCH = 1024
P = 2
Q = 10
GRID = NETS // (P * Q)


def _block_diag(w_stacked):
    zeros = jnp.zeros((D, D), dtype=w_stacked.dtype)
    rows = []
    for p in range(P):
        blk = w_stacked[p * D:(p + 1) * D, :]
        row = [blk if q == p else zeros for q in range(P)]
        rows.append(jnp.concatenate(row, axis=1))
    return jnp.concatenate(rows, axis=0)


def _mm(a, b):
    return jax.lax.dot_general(
        a, b, (((1,), (0,)), ((), ())),
        precision=jax.lax.Precision.DEFAULT,
        preferred_element_type=jnp.float32)


def _step(x_ref, w0_ref, w1_ref, w2_ref, out_ref):
    for q in range(Q):
        sl = pl.ds(q * P * D, P * D)
        b0 = _block_diag(w0_ref[sl, :])
        b1 = _block_diag(w1_ref[sl, :])
        b2 = _block_diag(w2_ref[sl, :])
        m = _mm(b2, _mm(b1, b0))
        out_ref[sl, :] = _mm(m.astype(jnp.bfloat16),
                             x_ref[sl, :].astype(jnp.bfloat16))


@jax.jit
def kernel(x, w0, w1, w2):
    w0m = w0.reshape(NETS * D, D)
    w1m = w1.reshape(NETS * D, D)
    w2m = w2.reshape(NETS * D, D)
    wspec = pl.BlockSpec((Q * P * D, D), lambda i: (i, 0))
    xspec = pl.BlockSpec((Q * P * D, BATCH), lambda i: (i, 0))
    return pl.pallas_call(
        _step,
        grid=(GRID,),
        in_specs=[xspec, wspec, wspec, wspec],
        out_specs=xspec,
        out_shape=jax.ShapeDtypeStruct((NETS * D, BATCH), jnp.float32),
    )(x, w0m, w1m, w2m)
